# Initial kernel scaffold; baseline (speedup 1.0000x reference)
import jax
import jax.numpy as jnp
from jax.experimental import pallas as pl


def kernel(x, edge_index, edge_weights, y_target, batch, W_emb, b_emb, W_mu, b_mu, gamma_mu, beta_mu, W_var, b_var, gamma_var, beta_var):
    raise NotImplementedError("write your pallas kernel here")


# speedup vs baseline: 1.0895x; 1.0895x over previous
import functools

import jax
import jax.numpy as jnp
from jax import lax
from jax.experimental import pallas as pl
from jax.experimental.pallas import tpu as pltpu

_N = 10000
_E = 160000
_D = 256
_H = 256
_Z = 128
_G = 64

_BLK = 1000
_NBLK = _N // _BLK


def _dense_body(batch_ref, y_ref, W_emb_ref, b_emb_ref,
                Wm_ref, wym_ref, bm_ref, gm_ref, betam_ref,
                Wv_ref, wyv_ref, bv_ref, gv_ref, betav_ref,
                agg_ref,
                zmu_ref, zlv_ref,
                pooled_acc, cnt_acc):
    i = pl.program_id(0)

    agg = agg_ref[...]
    emb = jnp.dot(agg, W_emb_ref[...],
                  preferred_element_type=jnp.float32) + b_emb_ref[...]
    emb = jnp.maximum(emb, 0.0)

    batch_blk = batch_ref[0, 0, :]
    gids = lax.broadcasted_iota(jnp.int32, (_G, _BLK), 0)
    onehot = (gids == batch_blk[None, :]).astype(jnp.float32)
    part = jnp.dot(onehot, emb,
                   preferred_element_type=jnp.float32,
                   precision=lax.Precision.HIGHEST)
    cnt_part = jnp.sum(onehot, axis=1, keepdims=True)

    @pl.when(i == 0)
    def _init():
        pooled_acc[...] = jnp.zeros_like(pooled_acc)
        cnt_acc[...] = jnp.zeros_like(cnt_acc)

    pooled_acc[...] += part
    cnt_acc[...] += cnt_part

    @pl.when(i == _NBLK - 1)
    def _finalize():
        cnt = jnp.maximum(cnt_acc[...], 1.0)
        gemb = pooled_acc[...] / cnt
        y = y_ref[...]

        def _head(W, wy, b, gamma, beta):
            z = (jnp.dot(gemb, W, preferred_element_type=jnp.float32)
                 + y * wy
                 + b)
            mu = jnp.mean(z, axis=0, keepdims=True)
            zc = z - mu
            var = jnp.mean(zc * zc, axis=0, keepdims=True)
            zn = (z - mu) / jnp.sqrt(var + 1e-5) * gamma + beta
            return jnp.maximum(zn, 0.0)

        zmu_ref[...] = _head(Wm_ref[...], wym_ref[...], bm_ref[...],
                             gm_ref[...], betam_ref[...])
        zr = _head(Wv_ref[...], wyv_ref[...], bv_ref[...],
                   gv_ref[...], betav_ref[...])
        zlv_ref[...] = 1.0 / (1.0 + jnp.exp(-zr))


def _dense_stage(agg, batch_r, y_target, W_emb, b_emb,
                 Wm, wym, bm, gm, betam, Wv, wyv, bv, gv, betav):
    const = lambda *_: (0, 0)
    grid_spec = pltpu.PrefetchScalarGridSpec(
        num_scalar_prefetch=0,
        grid=(_NBLK,),
        in_specs=[
            pl.BlockSpec((1, 1, _BLK), lambda i: (i, 0, 0)),
            pl.BlockSpec((_G, 1), const),
            pl.BlockSpec((_D, _H), const),
            pl.BlockSpec((1, _H), const),
            pl.BlockSpec((_H, _Z), const),
            pl.BlockSpec((1, _Z), const),
            pl.BlockSpec((1, _Z), const),
            pl.BlockSpec((1, _Z), const),
            pl.BlockSpec((1, _Z), const),
            pl.BlockSpec((_H, _Z), const),
            pl.BlockSpec((1, _Z), const),
            pl.BlockSpec((1, _Z), const),
            pl.BlockSpec((1, _Z), const),
            pl.BlockSpec((1, _Z), const),
            pl.BlockSpec((_BLK, _D), lambda i: (i, 0)),
        ],
        out_specs=[
            pl.BlockSpec((_G, _Z), const),
            pl.BlockSpec((_G, _Z), const),
        ],
        scratch_shapes=[
            pltpu.VMEM((_G, _H), jnp.float32),
            pltpu.VMEM((_G, 1), jnp.float32),
        ],
    )
    return pl.pallas_call(
        _dense_body,
        grid_spec=grid_spec,
        out_shape=[
            jax.ShapeDtypeStruct((_G, _Z), jnp.float32),
            jax.ShapeDtypeStruct((_G, _Z), jnp.float32),
        ],
        compiler_params=pltpu.CompilerParams(
            dimension_semantics=("arbitrary",),
        ),
    )(batch_r, y_target, W_emb, b_emb,
      Wm, wym, bm, gm, betam, Wv, wyv, bv, gv, betav, agg)


def kernel(x, edge_index, edge_weights, y_target, batch,
           W_emb, b_emb, W_mu, b_mu, gamma_mu, beta_mu,
           W_var, b_var, gamma_var, beta_var):
    src = edge_index[0]
    dst = edge_index[1]
    msg = edge_weights[:, None] * jnp.take(x, src, axis=0)
    agg = jax.ops.segment_sum(msg, dst, num_segments=_N)

    batch_r = batch.reshape(_NBLK, 1, _BLK)
    zmu, zlv = _dense_stage(
        agg, batch_r, y_target, W_emb, b_emb.reshape(1, _H),
        W_mu[:_H], W_mu[_H:].reshape(1, _Z), b_mu.reshape(1, _Z),
        gamma_mu.reshape(1, _Z), beta_mu.reshape(1, _Z),
        W_var[:_H], W_var[_H:].reshape(1, _Z), b_var.reshape(1, _Z),
        gamma_var.reshape(1, _Z), beta_var.reshape(1, _Z))
    return (zmu, zlv)


# speedup vs baseline: 2.6236x; 2.4082x over previous
import functools

import jax
import jax.numpy as jnp
from jax import lax
from jax.experimental import pallas as pl
from jax.experimental.pallas import tpu as pltpu
from jax.experimental.pallas import tpu_sc as plsc

_N = 10000
_E = 160000
_D = 256
_H = 256
_Z = 128
_G = 64

_NTILES = 16
_NPAD = 10240
_BAND = _NPAD // _NTILES
_EPAD = 163840
_CHUNK = 256
_NCHUNK = _EPAD // _NTILES // _CHUNK
_EROWS = _EPAD // 128

_BLK = 400
_NBLK = _N // _BLK


def _sc_body(xi_ref, srcidx_ref, dst_ref, w_ref, out_ref,
             src_v, dst_v, w_v, rows_v, sem, acc):
    c = lax.axis_index("c")
    s = lax.axis_index("s")
    b0 = s * _BAND

    def _zrow(i, _):
        for k in range(8):
            rows_v[0, i, pl.ds(k * 16, 16)] = jnp.zeros((16,), jnp.float32)
        return 0
    lax.fori_loop(0, 128, _zrow, 0)
    for q in range(_BAND // 128):
        pltpu.sync_copy(rows_v.at[0], acc.at[pl.ds(b0 + q * 128, 128)])
    plsc.subcore_barrier()

    def _chunk(k, _):
        r = s * (_EROWS // _NTILES) + k * (_CHUNK // 128)
        pltpu.sync_copy(srcidx_ref.at[c, pl.ds(r, 2)], src_v)
        pltpu.sync_copy(dst_ref.at[pl.ds(r, 2)], dst_v)
        pltpu.sync_copy(w_ref.at[pl.ds(r, 2)], w_v)
        for j in range(2):
            pltpu.async_copy(xi_ref.at[src_v.at[j]], rows_v.at[j], sem).wait()
        for j in range(2):
            def _scale(g, _):
                w16 = w_v[j, pl.ds(g * 16, 16)]
                for k in range(16):
                    w = w16[k]
                    row = g * 16 + k
                    for k8 in range(8):
                        sl = pl.ds(k8 * 16, 16)
                        rows_v[j, row, sl] = rows_v[j, row, sl] * w
                return 0
            lax.fori_loop(0, 8, _scale, 0)
        for j in range(2):
            pltpu.sync_copy(rows_v.at[j], acc.at[dst_v.at[j]], add=True)
        return 0

    lax.fori_loop(0, _NCHUNK, _chunk, 0)
    plsc.subcore_barrier()
    pltpu.sync_copy(acc.at[pl.ds(b0, _BAND)], out_ref.at[c, pl.ds(b0, _BAND)])


def _sc_stage(x_i, srcidx2, dst2, w2):
    mesh = plsc.VectorSubcoreMesh(core_axis_name="c", subcore_axis_name="s")
    return pl.kernel(
        _sc_body,
        out_type=jax.ShapeDtypeStruct((2, _NPAD, 128), jnp.float32),
        mesh=mesh,
        scratch_types=[
            pltpu.VMEM((2, 128), jnp.int32),
            pltpu.VMEM((2, 128), jnp.int32),
            pltpu.VMEM((2, 128), jnp.float32),
            pltpu.VMEM((2, 128, 128), jnp.float32),
            pltpu.SemaphoreType.DMA,
            pltpu.VMEM_SHARED((_NPAD, 128), jnp.float32),
        ],
    )(x_i, srcidx2, dst2, w2)


def _dense_body(batch_ref, y_ref, W_emb_ref, b_emb_ref,
                Wm_ref, wym_ref, bm_ref, gm_ref, betam_ref,
                Wv_ref, wyv_ref, bv_ref, gv_ref, betav_ref,
                agg_ref,
                zmu_ref, zlv_ref,
                pooled_acc, cnt_acc):
    i = pl.program_id(0)

    agg = jnp.concatenate([agg_ref[0], agg_ref[1]], axis=-1)
    emb = jnp.dot(agg, W_emb_ref[...],
                  preferred_element_type=jnp.float32) + b_emb_ref[...]
    emb = jnp.maximum(emb, 0.0)

    batch_blk = batch_ref[0, 0, :]
    gids = lax.broadcasted_iota(jnp.int32, (_G, _BLK), 0)
    onehot = (gids == batch_blk[None, :]).astype(jnp.float32)
    part = jnp.dot(onehot, emb,
                   preferred_element_type=jnp.float32,
                   precision=lax.Precision.HIGHEST)
    cnt_part = jnp.sum(onehot, axis=1, keepdims=True)

    @pl.when(i == 0)
    def _init():
        pooled_acc[...] = jnp.zeros_like(pooled_acc)
        cnt_acc[...] = jnp.zeros_like(cnt_acc)

    pooled_acc[...] += part
    cnt_acc[...] += cnt_part

    @pl.when(i == _NBLK - 1)
    def _finalize():
        cnt = jnp.maximum(cnt_acc[...], 1.0)
        gemb = pooled_acc[...] / cnt
        y = y_ref[...]

        def _head(W, wy, b, gamma, beta):
            z = (jnp.dot(gemb, W, preferred_element_type=jnp.float32)
                 + y * wy
                 + b)
            mu = jnp.mean(z, axis=0, keepdims=True)
            zc = z - mu
            var = jnp.mean(zc * zc, axis=0, keepdims=True)
            zn = (z - mu) / jnp.sqrt(var + 1e-5) * gamma + beta
            return jnp.maximum(zn, 0.0)

        zmu_ref[...] = _head(Wm_ref[...], wym_ref[...], bm_ref[...],
                             gm_ref[...], betam_ref[...])
        zr = _head(Wv_ref[...], wyv_ref[...], bv_ref[...],
                   gv_ref[...], betav_ref[...])
        zlv_ref[...] = 1.0 / (1.0 + jnp.exp(-zr))


def _dense_stage(agg2, batch_r, y_target, W_emb, b_emb,
                 Wm, wym, bm, gm, betam, Wv, wyv, bv, gv, betav):
    const = lambda *_: (0, 0)
    grid_spec = pltpu.PrefetchScalarGridSpec(
        num_scalar_prefetch=0,
        grid=(_NBLK,),
        in_specs=[
            pl.BlockSpec((1, 1, _BLK), lambda i: (i, 0, 0)),
            pl.BlockSpec((_G, 1), const),
            pl.BlockSpec((_D, _H), const),
            pl.BlockSpec((1, _H), const),
            pl.BlockSpec((_H, _Z), const),
            pl.BlockSpec((1, _Z), const),
            pl.BlockSpec((1, _Z), const),
            pl.BlockSpec((1, _Z), const),
            pl.BlockSpec((1, _Z), const),
            pl.BlockSpec((_H, _Z), const),
            pl.BlockSpec((1, _Z), const),
            pl.BlockSpec((1, _Z), const),
            pl.BlockSpec((1, _Z), const),
            pl.BlockSpec((1, _Z), const),
            pl.BlockSpec((2, _BLK, 128), lambda i: (0, i, 0)),
        ],
        out_specs=[
            pl.BlockSpec((_G, _Z), const),
            pl.BlockSpec((_G, _Z), const),
        ],
        scratch_shapes=[
            pltpu.VMEM((_G, _H), jnp.float32),
            pltpu.VMEM((_G, 1), jnp.float32),
        ],
    )
    return pl.pallas_call(
        _dense_body,
        grid_spec=grid_spec,
        out_shape=[
            jax.ShapeDtypeStruct((_G, _Z), jnp.float32),
            jax.ShapeDtypeStruct((_G, _Z), jnp.float32),
        ],
        compiler_params=pltpu.CompilerParams(
            dimension_semantics=("arbitrary",),
        ),
    )(batch_r, y_target, W_emb, b_emb,
      Wm, wym, bm, gm, betam, Wv, wyv, bv, gv, betav, agg2)


def kernel(x, edge_index, edge_weights, y_target, batch,
           W_emb, b_emb, W_mu, b_mu, gamma_mu, beta_mu,
           W_var, b_var, gamma_var, beta_var):
    src = edge_index[0]
    dst = edge_index[1]
    pad = _EPAD - _E
    src_p = jnp.pad(src, (0, pad))
    dst_p = jnp.pad(dst, (0, pad))
    w_p = jnp.pad(edge_weights, (0, pad))

    x_i = x.reshape(_N, 2, 128).reshape(2 * _N, 128)
    srcidx2 = jnp.stack([src_p * 2, src_p * 2 + 1]).reshape(2, _EROWS, 128)
    dst2 = dst_p.reshape(_EROWS, 128)
    w2 = w_p.reshape(_EROWS, 128)

    agg2 = _sc_stage(x_i, srcidx2, dst2, w2)

    batch_r = batch.reshape(_NBLK, 1, _BLK)
    zmu, zlv = _dense_stage(
        agg2, batch_r, y_target, W_emb, b_emb.reshape(1, _H),
        W_mu[:_H], W_mu[_H:].reshape(1, _Z), b_mu.reshape(1, _Z),
        gamma_mu.reshape(1, _Z), beta_mu.reshape(1, _Z),
        W_var[:_H], W_var[_H:].reshape(1, _Z), b_var.reshape(1, _Z),
        gamma_var.reshape(1, _Z), beta_var.reshape(1, _Z))
    return (zmu, zlv)


# speedup vs baseline: 3.5645x; 1.3586x over previous
import functools

import jax
import jax.numpy as jnp
from jax import lax
from jax.experimental import pallas as pl
from jax.experimental.pallas import tpu as pltpu
from jax.experimental.pallas import tpu_sc as plsc

_N = 10000
_E = 160000
_D = 256
_H = 256
_Z = 128
_G = 64

_NTILES = 16
_NPAD = 10240
_BAND = _NPAD // _NTILES
_EPAD = 163840
_CHUNK = 256
_NCHUNK = _EPAD // _NTILES // _CHUNK
_EROWS = _EPAD // 128

_BLK = 400
_NBLK = _N // _BLK


_UEDGE = 64
_UNITS = _EPAD // _NTILES // _UEDGE
_PUNITS = _UNITS // 2
_UROWS = _EPAD // _UEDGE


def _sc_body(xi_ref, srcidx_ref, dst_ref, w_ref, out_ref,
             src_all, dst_all, w_all, rows_v,
             sem_g0, sem_g1, sem_s0, sem_s1, acc):
    c = lax.axis_index("c")
    s = lax.axis_index("s")
    b0 = s * _BAND
    sem_g = (sem_g0, sem_g1)
    sem_s = (sem_s0, sem_s1)

    def _zrow(i, _):
        for k in range(8):
            rows_v[0, i, pl.ds(k * 16, 16)] = jnp.zeros((16,), jnp.float32)
        return 0
    lax.fori_loop(0, _UEDGE, _zrow, 0)
    for q in range(_BAND // _UEDGE):
        pltpu.sync_copy(rows_v.at[0], acc.at[pl.ds(b0 + q * _UEDGE, _UEDGE)])
    plsc.subcore_barrier()

    def _wait_gather(b):
        pltpu.make_async_copy(
            xi_ref.at[src_all.at[0]], rows_v.at[b], sem_g[b]).wait()

    def _wait_scatter(b):
        pltpu.make_async_copy(
            rows_v.at[b], acc.at[dst_all.at[0]], sem_s[b]).wait()

    for phase in range(2):
        base = s * _UNITS + phase * _PUNITS
        pltpu.sync_copy(srcidx_ref.at[c, pl.ds(base, _PUNITS)], src_all)
        pltpu.sync_copy(dst_ref.at[pl.ds(base, _PUNITS)], dst_all)
        pltpu.sync_copy(w_ref.at[pl.ds(base, _PUNITS)], w_all)

        pltpu.async_copy(xi_ref.at[src_all.at[0]], rows_v.at[0], sem_g[0])

        def _outer(kk, _):
            for b in range(2):
                u = 2 * kk + b

                @pl.when(u + 1 < _PUNITS)
                def _fire_next():
                    @pl.when(u >= 1)
                    def _():
                        _wait_scatter(1 - b)
                    pltpu.async_copy(xi_ref.at[src_all.at[u + 1]],
                                     rows_v.at[1 - b], sem_g[1 - b])

                _wait_gather(b)

                def _scale(g, _):
                    w16 = w_all[u, pl.ds(g * 16, 16)]
                    for kx in range(16):
                        w = w16[kx]
                        row = g * 16 + kx
                        for k8 in range(8):
                            sl = pl.ds(k8 * 16, 16)
                            rows_v[b, row, sl] = rows_v[b, row, sl] * w
                    return 0
                lax.fori_loop(0, _UEDGE // 16, _scale, 0)

                pltpu.async_copy(rows_v.at[b], acc.at[dst_all.at[u]],
                                 sem_s[b], add=True)
            return 0

        lax.fori_loop(0, _PUNITS // 2, _outer, 0)
        _wait_scatter(0)
        _wait_scatter(1)
    plsc.subcore_barrier()
    pltpu.sync_copy(acc.at[pl.ds(b0, _BAND)], out_ref.at[c, pl.ds(b0, _BAND)])


def _sc_stage(x_i, srcidx2, dst2, w2):
    mesh = plsc.VectorSubcoreMesh(core_axis_name="c", subcore_axis_name="s")
    return pl.kernel(
        _sc_body,
        out_type=jax.ShapeDtypeStruct((2, _NPAD, 128), jnp.float32),
        mesh=mesh,
        scratch_types=[
            pltpu.VMEM((_PUNITS, _UEDGE), jnp.int32),
            pltpu.VMEM((_PUNITS, _UEDGE), jnp.int32),
            pltpu.VMEM((_PUNITS, _UEDGE), jnp.float32),
            pltpu.VMEM((2, _UEDGE, 128), jnp.float32),
            pltpu.SemaphoreType.DMA,
            pltpu.SemaphoreType.DMA,
            pltpu.SemaphoreType.DMA,
            pltpu.SemaphoreType.DMA,
            pltpu.VMEM_SHARED((_NPAD, 128), jnp.float32),
        ],
    )(x_i, srcidx2, dst2, w2)


def _dense_body(batch_ref, y_ref, W_emb_ref, b_emb_ref,
                Wm_ref, wym_ref, bm_ref, gm_ref, betam_ref,
                Wv_ref, wyv_ref, bv_ref, gv_ref, betav_ref,
                agg_ref,
                zmu_ref, zlv_ref,
                pooled_acc, cnt_acc):
    i = pl.program_id(0)

    agg = jnp.concatenate([agg_ref[0], agg_ref[1]], axis=-1)
    emb = jnp.dot(agg, W_emb_ref[...],
                  preferred_element_type=jnp.float32) + b_emb_ref[...]
    emb = jnp.maximum(emb, 0.0)

    batch_blk = batch_ref[0, 0, :]
    gids = lax.broadcasted_iota(jnp.int32, (_G, _BLK), 0)
    onehot = (gids == batch_blk[None, :]).astype(jnp.float32)
    part = jnp.dot(onehot, emb,
                   preferred_element_type=jnp.float32,
                   precision=lax.Precision.HIGHEST)
    cnt_part = jnp.sum(onehot, axis=1, keepdims=True)

    @pl.when(i == 0)
    def _init():
        pooled_acc[...] = jnp.zeros_like(pooled_acc)
        cnt_acc[...] = jnp.zeros_like(cnt_acc)

    pooled_acc[...] += part
    cnt_acc[...] += cnt_part

    @pl.when(i == _NBLK - 1)
    def _finalize():
        cnt = jnp.maximum(cnt_acc[...], 1.0)
        gemb = pooled_acc[...] / cnt
        y = y_ref[...]

        def _head(W, wy, b, gamma, beta):
            z = (jnp.dot(gemb, W, preferred_element_type=jnp.float32)
                 + y * wy
                 + b)
            mu = jnp.mean(z, axis=0, keepdims=True)
            zc = z - mu
            var = jnp.mean(zc * zc, axis=0, keepdims=True)
            zn = (z - mu) / jnp.sqrt(var + 1e-5) * gamma + beta
            return jnp.maximum(zn, 0.0)

        zmu_ref[...] = _head(Wm_ref[...], wym_ref[...], bm_ref[...],
                             gm_ref[...], betam_ref[...])
        zr = _head(Wv_ref[...], wyv_ref[...], bv_ref[...],
                   gv_ref[...], betav_ref[...])
        zlv_ref[...] = 1.0 / (1.0 + jnp.exp(-zr))


def _dense_stage(agg2, batch_r, y_target, W_emb, b_emb,
                 Wm, wym, bm, gm, betam, Wv, wyv, bv, gv, betav):
    const = lambda *_: (0, 0)
    grid_spec = pltpu.PrefetchScalarGridSpec(
        num_scalar_prefetch=0,
        grid=(_NBLK,),
        in_specs=[
            pl.BlockSpec((1, 1, _BLK), lambda i: (i, 0, 0)),
            pl.BlockSpec((_G, 1), const),
            pl.BlockSpec((_D, _H), const),
            pl.BlockSpec((1, _H), const),
            pl.BlockSpec((_H, _Z), const),
            pl.BlockSpec((1, _Z), const),
            pl.BlockSpec((1, _Z), const),
            pl.BlockSpec((1, _Z), const),
            pl.BlockSpec((1, _Z), const),
            pl.BlockSpec((_H, _Z), const),
            pl.BlockSpec((1, _Z), const),
            pl.BlockSpec((1, _Z), const),
            pl.BlockSpec((1, _Z), const),
            pl.BlockSpec((1, _Z), const),
            pl.BlockSpec((2, _BLK, 128), lambda i: (0, i, 0)),
        ],
        out_specs=[
            pl.BlockSpec((_G, _Z), const),
            pl.BlockSpec((_G, _Z), const),
        ],
        scratch_shapes=[
            pltpu.VMEM((_G, _H), jnp.float32),
            pltpu.VMEM((_G, 1), jnp.float32),
        ],
    )
    return pl.pallas_call(
        _dense_body,
        grid_spec=grid_spec,
        out_shape=[
            jax.ShapeDtypeStruct((_G, _Z), jnp.float32),
            jax.ShapeDtypeStruct((_G, _Z), jnp.float32),
        ],
        compiler_params=pltpu.CompilerParams(
            dimension_semantics=("arbitrary",),
        ),
    )(batch_r, y_target, W_emb, b_emb,
      Wm, wym, bm, gm, betam, Wv, wyv, bv, gv, betav, agg2)


def kernel(x, edge_index, edge_weights, y_target, batch,
           W_emb, b_emb, W_mu, b_mu, gamma_mu, beta_mu,
           W_var, b_var, gamma_var, beta_var):
    src = edge_index[0]
    dst = edge_index[1]
    pad = _EPAD - _E
    src_p = jnp.pad(src, (0, pad))
    dst_p = jnp.pad(dst, (0, pad))
    w_p = jnp.pad(edge_weights, (0, pad))

    x_i = x.reshape(_N, 2, 128).reshape(2 * _N, 128)
    srcidx2 = jnp.stack([src_p * 2, src_p * 2 + 1]).reshape(2, _UROWS, _UEDGE)
    dst2 = dst_p.reshape(_UROWS, _UEDGE)
    w2 = w_p.reshape(_UROWS, _UEDGE)

    agg2 = _sc_stage(x_i, srcidx2, dst2, w2)

    batch_r = batch.reshape(_NBLK, 1, _BLK)
    zmu, zlv = _dense_stage(
        agg2, batch_r, y_target, W_emb, b_emb.reshape(1, _H),
        W_mu[:_H], W_mu[_H:].reshape(1, _Z), b_mu.reshape(1, _Z),
        gamma_mu.reshape(1, _Z), beta_mu.reshape(1, _Z),
        W_var[:_H], W_var[_H:].reshape(1, _Z), b_var.reshape(1, _Z),
        gamma_var.reshape(1, _Z), beta_var.reshape(1, _Z))
    return (zmu, zlv)


# speedup vs baseline: 3.5915x; 1.0076x over previous
import functools

import jax
import jax.numpy as jnp
from jax import lax
from jax.experimental import pallas as pl
from jax.experimental.pallas import tpu as pltpu
from jax.experimental.pallas import tpu_sc as plsc

_N = 10000
_E = 160000
_D = 256
_H = 256
_Z = 128
_G = 64

_NTILES = 16
_NPAD = 10240
_BAND = _NPAD // _NTILES
_EPAD = 163840
_CHUNK = 256
_NCHUNK = _EPAD // _NTILES // _CHUNK
_EROWS = _EPAD // 128

_BLK = 400
_NBLK = _N // _BLK


_UEDGE = 64
_UNITS = _EPAD // _NTILES // _UEDGE
_NBUF = 4
_NPHASE = 4
_PUNITS = _UNITS // _NPHASE
_UROWS = _EPAD // _UEDGE


def _sc_body(xi_ref, srcidx_ref, dst_ref, w_ref, out_ref,
             src_all, dst_all, w_all, rows_v,
             sem_g0, sem_g1, sem_g2, sem_g3,
             sem_s0, sem_s1, sem_s2, sem_s3, acc):
    c = lax.axis_index("c")
    s = lax.axis_index("s")
    b0 = s * _BAND
    sem_g = (sem_g0, sem_g1, sem_g2, sem_g3)
    sem_s = (sem_s0, sem_s1, sem_s2, sem_s3)

    def _zrow(i, _):
        for k in range(8):
            rows_v[0, i, pl.ds(k * 16, 16)] = jnp.zeros((16,), jnp.float32)
        return 0
    lax.fori_loop(0, _UEDGE, _zrow, 0)
    for q in range(_BAND // _UEDGE):
        pltpu.sync_copy(rows_v.at[0], acc.at[pl.ds(b0 + q * _UEDGE, _UEDGE)])
    plsc.subcore_barrier()

    def _wait_gather(b):
        pltpu.make_async_copy(
            xi_ref.at[src_all.at[0]], rows_v.at[b], sem_g[b]).wait()

    def _wait_scatter(b):
        pltpu.make_async_copy(
            rows_v.at[b], acc.at[dst_all.at[0]], sem_s[b]).wait()

    for phase in range(_NPHASE):
        base = s * _UNITS + phase * _PUNITS
        pltpu.sync_copy(srcidx_ref.at[c, pl.ds(base, _PUNITS)], src_all)
        pltpu.sync_copy(dst_ref.at[pl.ds(base, _PUNITS)], dst_all)
        pltpu.sync_copy(w_ref.at[pl.ds(base, _PUNITS)], w_all)

        for b in range(_NBUF - 1):
            pltpu.async_copy(xi_ref.at[src_all.at[b]], rows_v.at[b],
                             sem_g[b])

        def _outer(kk, _):
            for b in range(_NBUF):
                u = _NBUF * kk + b

                _wait_gather(b)

                def _scale(g, _):
                    w16 = w_all[u, pl.ds(g * 16, 16)]
                    for kx in range(16):
                        w = w16[kx]
                        row = g * 16 + kx
                        for k8 in range(8):
                            sl = pl.ds(k8 * 16, 16)
                            rows_v[b, row, sl] = rows_v[b, row, sl] * w
                    return 0
                lax.fori_loop(0, _UEDGE // 16, _scale, 0)

                pltpu.async_copy(rows_v.at[b], acc.at[dst_all.at[u]],
                                 sem_s[b], add=True)

                nb = (b + _NBUF - 1) % _NBUF
                @pl.when(u + _NBUF - 1 < _PUNITS)
                def _fire_next():
                    @pl.when(u >= 1)
                    def _():
                        _wait_scatter(nb)
                    pltpu.async_copy(
                        xi_ref.at[src_all.at[u + _NBUF - 1]],
                        rows_v.at[nb], sem_g[nb])
            return 0

        lax.fori_loop(0, _PUNITS // _NBUF, _outer, 0)
        for b in range(_NBUF):
            _wait_scatter(b)
    plsc.subcore_barrier()
    pltpu.sync_copy(acc.at[pl.ds(b0, _BAND)], out_ref.at[c, pl.ds(b0, _BAND)])


def _sc_stage(x_i, srcidx2, dst2, w2):
    mesh = plsc.VectorSubcoreMesh(core_axis_name="c", subcore_axis_name="s")
    return pl.kernel(
        _sc_body,
        out_type=jax.ShapeDtypeStruct((2, _NPAD, 128), jnp.float32),
        mesh=mesh,
        scratch_types=[
            pltpu.VMEM((_PUNITS, _UEDGE), jnp.int32),
            pltpu.VMEM((_PUNITS, _UEDGE), jnp.int32),
            pltpu.VMEM((_PUNITS, _UEDGE), jnp.float32),
            pltpu.VMEM((_NBUF, _UEDGE, 128), jnp.float32),
            pltpu.SemaphoreType.DMA,
            pltpu.SemaphoreType.DMA,
            pltpu.SemaphoreType.DMA,
            pltpu.SemaphoreType.DMA,
            pltpu.SemaphoreType.DMA,
            pltpu.SemaphoreType.DMA,
            pltpu.SemaphoreType.DMA,
            pltpu.SemaphoreType.DMA,
            pltpu.VMEM_SHARED((_NPAD, 128), jnp.float32),
        ],
    )(x_i, srcidx2, dst2, w2)


def _dense_body(batch_ref, y_ref, W_emb_ref, b_emb_ref,
                Wm_ref, wym_ref, bm_ref, gm_ref, betam_ref,
                Wv_ref, wyv_ref, bv_ref, gv_ref, betav_ref,
                agg_ref,
                zmu_ref, zlv_ref,
                pooled_acc, cnt_acc):
    i = pl.program_id(0)

    agg = jnp.concatenate([agg_ref[0], agg_ref[1]], axis=-1)
    emb = jnp.dot(agg, W_emb_ref[...],
                  preferred_element_type=jnp.float32) + b_emb_ref[...]
    emb = jnp.maximum(emb, 0.0)

    batch_blk = batch_ref[0, 0, :]
    gids = lax.broadcasted_iota(jnp.int32, (_G, _BLK), 0)
    onehot = (gids == batch_blk[None, :]).astype(jnp.float32)
    part = jnp.dot(onehot, emb,
                   preferred_element_type=jnp.float32,
                   precision=lax.Precision.HIGHEST)
    cnt_part = jnp.sum(onehot, axis=1, keepdims=True)

    @pl.when(i == 0)
    def _init():
        pooled_acc[...] = jnp.zeros_like(pooled_acc)
        cnt_acc[...] = jnp.zeros_like(cnt_acc)

    pooled_acc[...] += part
    cnt_acc[...] += cnt_part

    @pl.when(i == _NBLK - 1)
    def _finalize():
        cnt = jnp.maximum(cnt_acc[...], 1.0)
        gemb = pooled_acc[...] / cnt
        y = y_ref[...]

        def _head(W, wy, b, gamma, beta):
            z = (jnp.dot(gemb, W, preferred_element_type=jnp.float32)
                 + y * wy
                 + b)
            mu = jnp.mean(z, axis=0, keepdims=True)
            zc = z - mu
            var = jnp.mean(zc * zc, axis=0, keepdims=True)
            zn = (z - mu) / jnp.sqrt(var + 1e-5) * gamma + beta
            return jnp.maximum(zn, 0.0)

        zmu_ref[...] = _head(Wm_ref[...], wym_ref[...], bm_ref[...],
                             gm_ref[...], betam_ref[...])
        zr = _head(Wv_ref[...], wyv_ref[...], bv_ref[...],
                   gv_ref[...], betav_ref[...])
        zlv_ref[...] = 1.0 / (1.0 + jnp.exp(-zr))


def _dense_stage(agg2, batch_r, y_target, W_emb, b_emb,
                 Wm, wym, bm, gm, betam, Wv, wyv, bv, gv, betav):
    const = lambda *_: (0, 0)
    grid_spec = pltpu.PrefetchScalarGridSpec(
        num_scalar_prefetch=0,
        grid=(_NBLK,),
        in_specs=[
            pl.BlockSpec((1, 1, _BLK), lambda i: (i, 0, 0)),
            pl.BlockSpec((_G, 1), const),
            pl.BlockSpec((_D, _H), const),
            pl.BlockSpec((1, _H), const),
            pl.BlockSpec((_H, _Z), const),
            pl.BlockSpec((1, _Z), const),
            pl.BlockSpec((1, _Z), const),
            pl.BlockSpec((1, _Z), const),
            pl.BlockSpec((1, _Z), const),
            pl.BlockSpec((_H, _Z), const),
            pl.BlockSpec((1, _Z), const),
            pl.BlockSpec((1, _Z), const),
            pl.BlockSpec((1, _Z), const),
            pl.BlockSpec((1, _Z), const),
            pl.BlockSpec((2, _BLK, 128), lambda i: (0, i, 0)),
        ],
        out_specs=[
            pl.BlockSpec((_G, _Z), const),
            pl.BlockSpec((_G, _Z), const),
        ],
        scratch_shapes=[
            pltpu.VMEM((_G, _H), jnp.float32),
            pltpu.VMEM((_G, 1), jnp.float32),
        ],
    )
    return pl.pallas_call(
        _dense_body,
        grid_spec=grid_spec,
        out_shape=[
            jax.ShapeDtypeStruct((_G, _Z), jnp.float32),
            jax.ShapeDtypeStruct((_G, _Z), jnp.float32),
        ],
        compiler_params=pltpu.CompilerParams(
            dimension_semantics=("arbitrary",),
        ),
    )(batch_r, y_target, W_emb, b_emb,
      Wm, wym, bm, gm, betam, Wv, wyv, bv, gv, betav, agg2)


def kernel(x, edge_index, edge_weights, y_target, batch,
           W_emb, b_emb, W_mu, b_mu, gamma_mu, beta_mu,
           W_var, b_var, gamma_var, beta_var):
    src = edge_index[0]
    dst = edge_index[1]
    pad = _EPAD - _E
    src_p = jnp.pad(src, (0, pad))
    dst_p = jnp.pad(dst, (0, pad))
    w_p = jnp.pad(edge_weights, (0, pad))

    x_i = x.reshape(_N, 2, 128).reshape(2 * _N, 128)
    srcidx2 = jnp.stack([src_p * 2, src_p * 2 + 1]).reshape(2, _UROWS, _UEDGE)
    dst2 = dst_p.reshape(_UROWS, _UEDGE)
    w2 = w_p.reshape(_UROWS, _UEDGE)

    agg2 = _sc_stage(x_i, srcidx2, dst2, w2)

    batch_r = batch.reshape(_NBLK, 1, _BLK)
    zmu, zlv = _dense_stage(
        agg2, batch_r, y_target, W_emb, b_emb.reshape(1, _H),
        W_mu[:_H], W_mu[_H:].reshape(1, _Z), b_mu.reshape(1, _Z),
        gamma_mu.reshape(1, _Z), beta_mu.reshape(1, _Z),
        W_var[:_H], W_var[_H:].reshape(1, _Z), b_var.reshape(1, _Z),
        gamma_var.reshape(1, _Z), beta_var.reshape(1, _Z))
    return (zmu, zlv)
